# baseline (speedup 1.0000x reference)
import jax
import jax.numpy as jnp
from jax.experimental import pallas as pl
from jax.experimental.pallas import tpu as pltpu

N = 1024
H = 512
_NEG = -1e30


def _gat2_kernel(
    x_ref, adj_ref, w1_ref, as1_ref, ad1_ref, b1_ref,
    w2_ref, as2_ref, ad2_ref, b2_ref, out_ref,
    mask_scr, o1_scr,
):
    j = pl.program_id(0)
    half = jnp.where(j < 2, j, j - 2)
    c = w1_ref.shape[1]

    def layer1():
        h = jnp.dot(x_ref[...], w1_ref[...], preferred_element_type=jnp.float32)
        return h, as1_ref[...], ad1_ref[...], b1_ref[...]

    def layer2():
        h = jnp.dot(jnp.maximum(o1_scr[...], 0.0), w2_ref[...],
                    preferred_element_type=jnp.float32)
        return h, as2_ref[...], ad2_ref[...], b2_ref[...]

    h, a_s, a_d, b = jax.lax.cond(j < 2, layer1, layer2)
    s = jnp.sum(h * a_s, axis=1)
    d = jnp.sum(h * a_d, axis=1)
    d_half = jnp.where(
        half == 0,
        jax.lax.slice(d, (0,), (H,)),
        jax.lax.slice(d, (H,), (N,)),
    )

    def build_mask():
        rowi = jax.lax.broadcasted_iota(jnp.int32, (N, H), 0)
        coli = jax.lax.broadcasted_iota(jnp.int32, (N, H), 1) + half * H
        valid = jnp.logical_or(rowi == coli, adj_ref[...] != 0)
        return jnp.where(valid, 0.0, _NEG).astype(jnp.float32)

    def reuse_mask():
        return mask_scr[:, pl.ds(half * H, H)]

    mask_add = jax.lax.cond(j < 2, build_mask, reuse_mask)

    @pl.when(j < 2)
    def _():
        mask_scr[:, pl.ds(half * H, H)] = mask_add

    e = s[:, None] + d_half[None, :] + mask_add
    e = jnp.maximum(e, 0.2 * e)
    w = jnp.exp(e).astype(jnp.bfloat16)
    hb = jnp.concatenate(
        [h.astype(jnp.bfloat16), jnp.ones((N, 1), dtype=jnp.bfloat16)], axis=1)
    agg2 = jax.lax.dot_general(
        w, hb, (((0,), (0,)), ((), ())), preferred_element_type=jnp.float32
    )
    res = agg2[:, :c] * (1.0 / (agg2[:, c:c + 1] + 1e-16)) + b

    @pl.when(j < 2)
    def _():
        o1_scr[pl.ds(half * H, H), :] = res

    @pl.when(j >= 2)
    def _():
        out_ref[...] = res


def kernel(x, adj, W1, att_src1, att_dst1, b1, W2, att_src2, att_dst2, b2):
    fin = x.shape[1]
    hid = W1.shape[1]
    fout = W2.shape[1]
    full = lambda shape: pl.BlockSpec(shape, lambda j: (0, 0))
    return pl.pallas_call(
        _gat2_kernel,
        grid=(4,),
        out_shape=jax.ShapeDtypeStruct((N, fout), jnp.float32),
        in_specs=[
            full((N, fin)),
            pl.BlockSpec((N, H), lambda j: (0, jnp.minimum(j, 1))),
            full((fin, hid)), full((1, hid)), full((1, hid)), full((1, hid)),
            full((hid, fout)), full((1, fout)), full((1, fout)), full((1, fout)),
        ],
        out_specs=pl.BlockSpec((H, fout), lambda j: (jnp.maximum(j - 2, 0), 0)),
        scratch_shapes=[
            pltpu.VMEM((N, N), jnp.float32),
            pltpu.VMEM((N, hid), jnp.float32),
        ],
    )(
        x, adj,
        W1, att_src1[None, :], att_dst1[None, :], b1[None, :],
        W2, att_src2[None, :], att_dst2[None, :], b2[None, :],
    )


# speedup vs baseline: 1.7423x; 1.7423x over previous
import jax
import jax.numpy as jnp
from jax.experimental import pallas as pl

N = 1024
_NEG = -1e30


def _layer(h_in, W, a_src, a_dst, b, mask_add, ones_col):
    c = W.shape[1]
    h = jnp.dot(h_in, W, preferred_element_type=jnp.float32)
    s = jnp.sum(h * a_src, axis=1)
    d = jnp.sum(h * a_dst, axis=1)
    e = s[:, None] + d[None, :] + mask_add
    e = jnp.maximum(e, 0.2 * e)
    w = jnp.exp(e).astype(jnp.bfloat16)
    hb = jnp.concatenate([h.astype(jnp.bfloat16), ones_col], axis=1)
    agg2 = jax.lax.dot_general(
        w, hb, (((0,), (0,)), ((), ())), preferred_element_type=jnp.float32
    )
    return agg2[:, :c] * (1.0 / (agg2[:, c:c + 1] + 1e-16)) + b


def _gat2_kernel(
    x_ref, adj_ref, w1_ref, as1_ref, ad1_ref, b1_ref,
    w2_ref, as2_ref, ad2_ref, b2_ref, out_ref,
):
    adj = adj_ref[...]
    row = jax.lax.broadcasted_iota(jnp.int32, (N, N), 0)
    col = jax.lax.broadcasted_iota(jnp.int32, (N, N), 1)
    valid = jnp.logical_or(row == col, adj != 0)
    mask_add = jnp.where(valid, 0.0, _NEG).astype(jnp.float32)
    ones_col = jnp.ones((N, 1), dtype=jnp.bfloat16)

    h1 = _layer(x_ref[...], w1_ref[...], as1_ref[...], ad1_ref[...],
                b1_ref[...], mask_add, ones_col)
    h1 = jnp.maximum(h1, 0.0)
    out_ref[...] = _layer(h1, w2_ref[...], as2_ref[...], ad2_ref[...],
                          b2_ref[...], mask_add, ones_col)


def kernel(x, adj, W1, att_src1, att_dst1, b1, W2, att_src2, att_dst2, b2):
    fout = W2.shape[1]
    return pl.pallas_call(
        _gat2_kernel,
        out_shape=jax.ShapeDtypeStruct((N, fout), jnp.float32),
    )(
        x, adj,
        W1, att_src1[None, :], att_dst1[None, :], b1[None, :],
        W2, att_src2[None, :], att_dst2[None, :], b2[None, :],
    )
